# Initial kernel scaffold; baseline (speedup 1.0000x reference)
import jax
import jax.numpy as jnp
from jax.experimental import pallas as pl


def kernel(triangles):
    raise NotImplementedError("write your pallas kernel here")


# speedup vs baseline: 110.2894x; 110.2894x over previous
import functools

import jax
import jax.numpy as jnp
from jax import lax
from jax.experimental import pallas as pl
from jax.experimental.pallas import tpu as pltpu
from jax.experimental.pallas import tpu_sc as plsc

F = 5000
B = 2
K = 8
FP = 5120
NCH = FP // 16
NTILE = 32
RPT = 157
RV = NTILE * RPT
NPAIR = RPT * K
CANDW = 1264
NBCH = CANDW // 16
OUTW = 2528
OUTN = RPT * 16
PAD_VAL = 1e30
EPS = 1e-8


def _cross(ax, ay, az, bx, by, bz):
    return (ay * bz - az * by, az * bx - ax * bz, ax * by - ay * bx)


def _dot3(ax, ay, az, bx, by, bz):
    return ax * bx + ay * by + az * bz


def _interval(d0, d1, d2, p0, p1, p2):
    lo = jnp.full((16,), jnp.inf, jnp.float32)
    hi = jnp.full((16,), -jnp.inf, jnp.float32)
    d = (d0, d1, d2)
    p = (p0, p1, p2)
    for a, b in ((0, 1), (1, 2), (2, 0)):
        da, db = d[a], d[b]
        crossing = (da * db) < 0.0
        denom = da - db
        safe = jnp.where(jnp.abs(denom) > 1e-30, denom, 1.0)
        t = da / safe
        s = p[a] + t * (p[b] - p[a])
        lo = jnp.where(crossing, jnp.minimum(lo, s), lo)
        hi = jnp.where(crossing, jnp.maximum(hi, s), hi)
    return lo, hi


def _tri_tri_hit(v, u):
    v0 = v[0:3]; v1 = v[3:6]; v2 = v[6:9]
    u0 = u[0:3]; u1 = u[3:6]; u2 = u[6:9]
    e1 = [u1[c] - u0[c] for c in range(3)]
    e2 = [u2[c] - u0[c] for c in range(3)]
    n2 = _cross(*e1, *e2)
    d2 = -_dot3(*n2, *u0)
    dv0 = _dot3(*n2, *v0) + d2
    dv1 = _dot3(*n2, *v1) + d2
    dv2 = _dot3(*n2, *v2) + d2
    f1 = [v1[c] - v0[c] for c in range(3)]
    f2 = [v2[c] - v0[c] for c in range(3)]
    n1 = _cross(*f1, *f2)
    d1 = -_dot3(*n1, *v0)
    du0 = _dot3(*n1, *u0) + d1
    du1 = _dot3(*n1, *u1) + d1
    du2 = _dot3(*n1, *u2) + d1
    sep_v = ((dv0 > EPS) & (dv1 > EPS) & (dv2 > EPS)) | \
            ((dv0 < -EPS) & (dv1 < -EPS) & (dv2 < -EPS))
    sep_u = ((du0 > EPS) & (du1 > EPS) & (du2 > EPS)) | \
            ((du0 < -EPS) & (du1 < -EPS) & (du2 < -EPS))
    dd = _cross(*n1, *n2)
    coplanar = _dot3(*dd, *dd) <= EPS
    pv0 = _dot3(*dd, *v0); pv1 = _dot3(*dd, *v1); pv2 = _dot3(*dd, *v2)
    pu0 = _dot3(*dd, *u0); pu1 = _dot3(*dd, *u1); pu2 = _dot3(*dd, *u2)
    lo1, hi1 = _interval(dv0, dv1, dv2, pv0, pv1, pv2)
    lo2, hi2 = _interval(du0, du1, du2, pu0, pu1, pu2)
    seg = jnp.maximum(lo1, lo2) <= jnp.minimum(hi1, hi2)
    return (~sep_v) & (~sep_u) & (~coplanar) & seg


def _splat_i32(x):
    return jnp.full((16,), x, jnp.int32)


def _bvh_body(tri_hbm, out_hbm, *scratch):
    tri_v = scratch[0:9]
    aabb_v = scratch[9:15]
    cand_v = scratch[15]
    out_v = scratch[16]
    cid = lax.axis_index("c")
    sid = lax.axis_index("s")
    wid = sid * 2 + cid
    row_base = wid * RPT
    lanes = lax.broadcasted_iota(jnp.int32, (16,), 0)

    for b in range(B):
        for cc in range(9):
            pltpu.sync_copy(tri_hbm.at[pl.ds((b * 9 + cc) * FP, FP)], tri_v[cc])

        def aabb_body(c, carry):
            o = c * 16
            for d in range(3):
                a0 = tri_v[d][pl.ds(o, 16)]
                a1 = tri_v[3 + d][pl.ds(o, 16)]
                a2 = tri_v[6 + d][pl.ds(o, 16)]
                aabb_v[d][pl.ds(o, 16)] = jnp.minimum(jnp.minimum(a0, a1), a2)
                aabb_v[3 + d][pl.ds(o, 16)] = jnp.maximum(jnp.maximum(a0, a1), a2)
            return carry
        lax.fori_loop(0, NCH, aabb_body, 0)

        def init_cand(c, carry):
            cand_v[pl.ds(c * 16, 16)] = _splat_i32(-1)
            return carry
        lax.fori_loop(0, NBCH, init_cand, 0)

        def init_out(c, carry):
            out_v[pl.ds(c * 16, 16)] = _splat_i32(-1)
            return carry
        lax.fori_loop(0, OUTW // 16, init_out, 0)

        def row_broad(r, carry):
            i = row_base + r

            @pl.when(i < F)
            def _():
                iv = jnp.full((16,), i, jnp.int32)
                bmin = [plsc.load_gather(aabb_v[d], [iv]) for d in range(3)]
                bmax = [plsc.load_gather(aabb_v[3 + d], [iv]) for d in range(3)]
                slot_base = r * K

                def cond(st):
                    ch, cnt = st
                    return (cnt < K) & (ch < NCH)

                def wbody(st):
                    ch, cnt = st
                    o = ch * 16
                    jv = o + lanes
                    ov = jv != i
                    for d in range(3):
                        cmin = aabb_v[d][pl.ds(o, 16)]
                        cmax = aabb_v[3 + d][pl.ds(o, 16)]
                        ov = ov & (bmin[d] <= cmax) & (cmin <= bmax[d])
                    inc = ov.astype(jnp.int32)
                    pref = plsc.cumsum(inc)
                    pos = pref + (cnt - 1)
                    m = ov & (pos < K)
                    idxv = slot_base + jnp.clip(pos, 0, K - 1)
                    plsc.store_scatter(cand_v, [idxv], jv, mask=m)
                    return (ch + 1, cnt + jnp.sum(inc))

                lax.while_loop(cond, wbody, (jnp.int32(0), jnp.int32(0)))
            return carry
        lax.fori_loop(0, RPT, row_broad, 0)

        def pair_chunk(c, carry):
            p = c * 16 + lanes
            rloc = lax.shift_right_logical(p, 3)
            ig = row_base + rloc
            jj = cand_v[pl.ds(c * 16, 16)]
            valid = jj >= 0
            jc = jnp.maximum(jj, 0)
            t1 = [plsc.load_gather(tri_v[cc], [ig]) for cc in range(9)]
            t2 = [plsc.load_gather(tri_v[cc], [jc]) for cc in range(9)]
            ok = valid & _tri_tri_hit(t1, t2)
            pos0 = p * 2
            plsc.store_scatter(out_v, [pos0], ig, mask=ok)
            plsc.store_scatter(out_v, [pos0 + 1], jj, mask=ok)
            return carry
        lax.fori_loop(0, NBCH, pair_chunk, 0)

        start = (b * RV + row_base) * 16
        pltpu.sync_copy(out_v.at[pl.ds(0, OUTN)], out_hbm.at[pl.ds(start, OUTN)])


@jax.jit
def _bvh_sc(tri_soa):
    mesh = plsc.VectorSubcoreMesh(core_axis_name="c", subcore_axis_name="s")
    fn = functools.partial(
        pl.kernel,
        out_type=jax.ShapeDtypeStruct((B * RV * 16,), jnp.int32),
        mesh=mesh,
        compiler_params=pltpu.CompilerParams(
            use_tc_tiling_on_sc=False, needs_layout_passes=False),
        scratch_types=(
            [pltpu.VMEM((FP,), jnp.float32) for _ in range(9)]
            + [pltpu.VMEM((FP,), jnp.float32) for _ in range(6)]
            + [pltpu.VMEM((CANDW,), jnp.int32),
               pltpu.VMEM((OUTW,), jnp.int32)]
        ),
    )(_bvh_body)
    return fn(tri_soa)


def kernel(triangles):
    tri_t = jnp.transpose(triangles, (0, 2, 3, 1)).reshape(B, 9, F)
    tri_p = jnp.pad(tri_t, ((0, 0), (0, 0), (0, FP - F)),
                    constant_values=PAD_VAL)
    out_flat = _bvh_sc(tri_p.reshape(-1))
    out = out_flat.reshape(B, RV, K, 2)[:, :F]
    return out.reshape(B, F * K, 2)


# speedup vs baseline: 132.3209x; 1.1998x over previous
import functools

import jax
import jax.numpy as jnp
from jax import lax
from jax.experimental import pallas as pl
from jax.experimental.pallas import tpu as pltpu
from jax.experimental.pallas import tpu_sc as plsc

F = 5000
B = 2
K = 8
FP = 5120
NCH = FP // 16
NTILE = 32
RPT = 157
RV = NTILE * RPT
NPAIR = RPT * K
CANDW = 1264
NBCH = CANDW // 16
OUTW = 2528
OUTN = RPT * 16
LASTR = F - (NTILE - 1) * RPT
OUTL = LASTR * 16
PAD_VAL = 1e30
EPS = 1e-8


def _cross(ax, ay, az, bx, by, bz):
    return (ay * bz - az * by, az * bx - ax * bz, ax * by - ay * bx)


def _dot3(ax, ay, az, bx, by, bz):
    return ax * bx + ay * by + az * bz


def _interval(d0, d1, d2, p0, p1, p2):
    lo = jnp.full((16,), jnp.inf, jnp.float32)
    hi = jnp.full((16,), -jnp.inf, jnp.float32)
    d = (d0, d1, d2)
    p = (p0, p1, p2)
    for a, b in ((0, 1), (1, 2), (2, 0)):
        da, db = d[a], d[b]
        crossing = (da * db) < 0.0
        denom = da - db
        safe = jnp.where(jnp.abs(denom) > 1e-30, denom, 1.0)
        t = da / safe
        s = p[a] + t * (p[b] - p[a])
        lo = jnp.where(crossing, jnp.minimum(lo, s), lo)
        hi = jnp.where(crossing, jnp.maximum(hi, s), hi)
    return lo, hi


def _tri_tri_hit(v, u):
    v0 = v[0:3]; v1 = v[3:6]; v2 = v[6:9]
    u0 = u[0:3]; u1 = u[3:6]; u2 = u[6:9]
    e1 = [u1[c] - u0[c] for c in range(3)]
    e2 = [u2[c] - u0[c] for c in range(3)]
    n2 = _cross(*e1, *e2)
    d2 = -_dot3(*n2, *u0)
    dv0 = _dot3(*n2, *v0) + d2
    dv1 = _dot3(*n2, *v1) + d2
    dv2 = _dot3(*n2, *v2) + d2
    f1 = [v1[c] - v0[c] for c in range(3)]
    f2 = [v2[c] - v0[c] for c in range(3)]
    n1 = _cross(*f1, *f2)
    d1 = -_dot3(*n1, *v0)
    du0 = _dot3(*n1, *u0) + d1
    du1 = _dot3(*n1, *u1) + d1
    du2 = _dot3(*n1, *u2) + d1
    sep_v = ((dv0 > EPS) & (dv1 > EPS) & (dv2 > EPS)) | \
            ((dv0 < -EPS) & (dv1 < -EPS) & (dv2 < -EPS))
    sep_u = ((du0 > EPS) & (du1 > EPS) & (du2 > EPS)) | \
            ((du0 < -EPS) & (du1 < -EPS) & (du2 < -EPS))
    dd = _cross(*n1, *n2)
    coplanar = _dot3(*dd, *dd) <= EPS
    pv0 = _dot3(*dd, *v0); pv1 = _dot3(*dd, *v1); pv2 = _dot3(*dd, *v2)
    pu0 = _dot3(*dd, *u0); pu1 = _dot3(*dd, *u1); pu2 = _dot3(*dd, *u2)
    lo1, hi1 = _interval(dv0, dv1, dv2, pv0, pv1, pv2)
    lo2, hi2 = _interval(du0, du1, du2, pu0, pu1, pu2)
    seg = jnp.maximum(lo1, lo2) <= jnp.minimum(hi1, hi2)
    return (~sep_v) & (~sep_u) & (~coplanar) & seg


def _splat_i32(x):
    return jnp.full((16,), x, jnp.int32)


def _bvh_body(tri_hbm, out_hbm, *scratch):
    tri_v = scratch[0:9]
    aabb_v = scratch[9:15]
    cand_v = scratch[15]
    out_v = scratch[16]
    cid = lax.axis_index("c")
    sid = lax.axis_index("s")
    wid = sid * 2 + cid
    row_base = wid * RPT
    lanes = lax.broadcasted_iota(jnp.int32, (16,), 0)

    for b in range(B):
        for cc in range(9):
            pltpu.sync_copy(tri_hbm.at[pl.ds((b * 9 + cc) * FP, FP)], tri_v[cc])

        def aabb_body(c, carry):
            o = c * 16
            for d in range(3):
                a0 = tri_v[d][pl.ds(o, 16)]
                a1 = tri_v[3 + d][pl.ds(o, 16)]
                a2 = tri_v[6 + d][pl.ds(o, 16)]
                aabb_v[d][pl.ds(o, 16)] = jnp.minimum(jnp.minimum(a0, a1), a2)
                aabb_v[3 + d][pl.ds(o, 16)] = jnp.maximum(jnp.maximum(a0, a1), a2)
            return carry
        lax.fori_loop(0, NCH, aabb_body, 0)

        def init_cand(c, carry):
            cand_v[pl.ds(c * 16, 16)] = _splat_i32(-1)
            return carry
        lax.fori_loop(0, NBCH, init_cand, 0)

        def init_out(c, carry):
            out_v[pl.ds(c * 16, 16)] = _splat_i32(-1)
            return carry
        lax.fori_loop(0, OUTW // 16, init_out, 0)

        def row_broad(r, carry):
            i = row_base + r

            @pl.when(i < F)
            def _():
                iv = jnp.full((16,), i, jnp.int32)
                bmin = [plsc.load_gather(aabb_v[d], [iv]) for d in range(3)]
                bmax = [plsc.load_gather(aabb_v[3 + d], [iv]) for d in range(3)]
                slot_base = r * K

                def cond(st):
                    ch, cnt = st
                    return (cnt < K) & (ch < NCH)

                def wbody(st):
                    ch, cnt = st
                    o = ch * 16
                    jv = o + lanes
                    ov = jv != i
                    for d in range(3):
                        cmin = aabb_v[d][pl.ds(o, 16)]
                        cmax = aabb_v[3 + d][pl.ds(o, 16)]
                        ov = ov & (bmin[d] <= cmax) & (cmin <= bmax[d])
                    inc = ov.astype(jnp.int32)
                    pref = plsc.cumsum(inc)
                    pos = pref + (cnt - 1)
                    m = ov & (pos < K)
                    idxv = slot_base + jnp.clip(pos, 0, K - 1)
                    plsc.store_scatter(cand_v, [idxv], jv, mask=m)
                    return (ch + 1, cnt + jnp.sum(inc))

                lax.while_loop(cond, wbody, (jnp.int32(0), jnp.int32(0)))
            return carry
        lax.fori_loop(0, RPT, row_broad, 0)

        def pair_chunk(c, carry):
            p = c * 16 + lanes
            rloc = lax.shift_right_logical(p, 3)
            ig = row_base + rloc
            jj = cand_v[pl.ds(c * 16, 16)]
            valid = jj >= 0
            jc = jnp.maximum(jj, 0)
            t1 = [plsc.load_gather(tri_v[cc], [ig]) for cc in range(9)]
            t2 = [plsc.load_gather(tri_v[cc], [jc]) for cc in range(9)]
            ok = valid & _tri_tri_hit(t1, t2)
            pos0 = p * 2
            plsc.store_scatter(out_v, [pos0], ig, mask=ok)
            plsc.store_scatter(out_v, [pos0 + 1], jj, mask=ok)
            return carry
        lax.fori_loop(0, NBCH, pair_chunk, 0)

        start = (b * F + row_base) * 16

        @pl.when(wid < NTILE - 1)
        def _():
            pltpu.sync_copy(out_v.at[pl.ds(0, OUTN)],
                            out_hbm.at[pl.ds(start, OUTN)])

        @pl.when(wid == NTILE - 1)
        def _():
            pltpu.sync_copy(out_v.at[pl.ds(0, OUTL)],
                            out_hbm.at[pl.ds(start, OUTL)])


@jax.jit
def _bvh_sc(tri_soa):
    mesh = plsc.VectorSubcoreMesh(core_axis_name="c", subcore_axis_name="s")
    fn = functools.partial(
        pl.kernel,
        out_type=jax.ShapeDtypeStruct((B * F * 16,), jnp.int32),
        mesh=mesh,
        compiler_params=pltpu.CompilerParams(
            use_tc_tiling_on_sc=False, needs_layout_passes=False),
        scratch_types=(
            [pltpu.VMEM((FP,), jnp.float32) for _ in range(9)]
            + [pltpu.VMEM((FP,), jnp.float32) for _ in range(6)]
            + [pltpu.VMEM((CANDW,), jnp.int32),
               pltpu.VMEM((OUTW,), jnp.int32)]
        ),
    )(_bvh_body)
    return fn(tri_soa)


def kernel(triangles):
    tri_t = jnp.transpose(triangles, (0, 2, 3, 1)).reshape(B, 9, F)
    tri_p = jnp.pad(tri_t, ((0, 0), (0, 0), (0, FP - F)),
                    constant_values=PAD_VAL)
    out_flat = _bvh_sc(tri_p.reshape(-1))
    return out_flat.reshape(B, F * K, 2)


# speedup vs baseline: 149.4763x; 1.1296x over previous
import functools

import jax
import jax.numpy as jnp
from jax import lax
from jax.experimental import pallas as pl
from jax.experimental.pallas import tpu as pltpu
from jax.experimental.pallas import tpu_sc as plsc

F = 5000
B = 2
K = 8
FP = 5120
NCH = FP // 16
NTILE = 32
RPT = 157
RV = NTILE * RPT
NPAIR = RPT * K
CANDW = 1264
NBCH = CANDW // 16
OUTW = 2528
OUTN = RPT * 16
LASTR = F - (NTILE - 1) * RPT
OUTL = LASTR * 16
PAD_VAL = 1e30
EPS = 1e-8


def _cross(ax, ay, az, bx, by, bz):
    return (ay * bz - az * by, az * bx - ax * bz, ax * by - ay * bx)


def _dot3(ax, ay, az, bx, by, bz):
    return ax * bx + ay * by + az * bz


def _interval(d0, d1, d2, p0, p1, p2):
    lo = jnp.full((16,), jnp.inf, jnp.float32)
    hi = jnp.full((16,), -jnp.inf, jnp.float32)
    d = (d0, d1, d2)
    p = (p0, p1, p2)
    for a, b in ((0, 1), (1, 2), (2, 0)):
        da, db = d[a], d[b]
        crossing = (da * db) < 0.0
        denom = da - db
        safe = jnp.where(jnp.abs(denom) > 1e-30, denom, 1.0)
        t = da / safe
        s = p[a] + t * (p[b] - p[a])
        lo = jnp.where(crossing, jnp.minimum(lo, s), lo)
        hi = jnp.where(crossing, jnp.maximum(hi, s), hi)
    return lo, hi


def _tri_tri_hit(v, u):
    v0 = v[0:3]; v1 = v[3:6]; v2 = v[6:9]
    u0 = u[0:3]; u1 = u[3:6]; u2 = u[6:9]
    e1 = [u1[c] - u0[c] for c in range(3)]
    e2 = [u2[c] - u0[c] for c in range(3)]
    n2 = _cross(*e1, *e2)
    d2 = -_dot3(*n2, *u0)
    dv0 = _dot3(*n2, *v0) + d2
    dv1 = _dot3(*n2, *v1) + d2
    dv2 = _dot3(*n2, *v2) + d2
    f1 = [v1[c] - v0[c] for c in range(3)]
    f2 = [v2[c] - v0[c] for c in range(3)]
    n1 = _cross(*f1, *f2)
    d1 = -_dot3(*n1, *v0)
    du0 = _dot3(*n1, *u0) + d1
    du1 = _dot3(*n1, *u1) + d1
    du2 = _dot3(*n1, *u2) + d1
    sep_v = ((dv0 > EPS) & (dv1 > EPS) & (dv2 > EPS)) | \
            ((dv0 < -EPS) & (dv1 < -EPS) & (dv2 < -EPS))
    sep_u = ((du0 > EPS) & (du1 > EPS) & (du2 > EPS)) | \
            ((du0 < -EPS) & (du1 < -EPS) & (du2 < -EPS))
    dd = _cross(*n1, *n2)
    coplanar = _dot3(*dd, *dd) <= EPS
    pv0 = _dot3(*dd, *v0); pv1 = _dot3(*dd, *v1); pv2 = _dot3(*dd, *v2)
    pu0 = _dot3(*dd, *u0); pu1 = _dot3(*dd, *u1); pu2 = _dot3(*dd, *u2)
    lo1, hi1 = _interval(dv0, dv1, dv2, pv0, pv1, pv2)
    lo2, hi2 = _interval(du0, du1, du2, pu0, pu1, pu2)
    seg = jnp.maximum(lo1, lo2) <= jnp.minimum(hi1, hi2)
    return (~sep_v) & (~sep_u) & (~coplanar) & seg


def _splat_i32(x):
    return jnp.full((16,), x, jnp.int32)


def _bvh_body(tri_hbm, out3_hbm, *scratch):
    tri_v = scratch[0:9]
    aabb_v = scratch[9:15]
    cand_v = scratch[15]
    out_v = scratch[16]
    cid = lax.axis_index("c")
    sid = lax.axis_index("s")
    wid = sid * 2 + cid
    row_base = wid * RPT
    lanes = lax.broadcasted_iota(jnp.int32, (16,), 0)

    for b in range(B):
        for cc in range(9):
            pltpu.sync_copy(tri_hbm.at[pl.ds((b * 9 + cc) * FP, FP)], tri_v[cc])

        def aabb_body(c, carry):
            o = c * 16
            for d in range(3):
                a0 = tri_v[d][pl.ds(o, 16)]
                a1 = tri_v[3 + d][pl.ds(o, 16)]
                a2 = tri_v[6 + d][pl.ds(o, 16)]
                aabb_v[d][pl.ds(o, 16)] = jnp.minimum(jnp.minimum(a0, a1), a2)
                aabb_v[3 + d][pl.ds(o, 16)] = jnp.maximum(jnp.maximum(a0, a1), a2)
            return carry
        lax.fori_loop(0, NCH, aabb_body, 0)

        def init_cand(c, carry):
            cand_v[pl.ds(c * 16, 16)] = _splat_i32(-1)
            return carry
        lax.fori_loop(0, NBCH, init_cand, 0)

        def init_out(c, carry):
            rows = c * 8 + lax.shift_right_logical(lanes, 1)
            cols = lanes & 1
            plsc.store_scatter(out_v, [rows, cols], _splat_i32(-1))
            return carry
        lax.fori_loop(0, OUTW // 16, init_out, 0)

        def row_broad(r, carry):
            i = row_base + r

            @pl.when(i < F)
            def _():
                iv = jnp.full((16,), i, jnp.int32)
                bmin = [plsc.load_gather(aabb_v[d], [iv]) for d in range(3)]
                bmax = [plsc.load_gather(aabb_v[3 + d], [iv]) for d in range(3)]
                slot_base = r * K

                def cond(st):
                    ch, cnt = st
                    return (cnt < K) & (ch < NCH)

                def wbody(st):
                    ch, cnt = st
                    o = ch * 16
                    jv = o + lanes
                    ov = jv != i
                    for d in range(3):
                        cmin = aabb_v[d][pl.ds(o, 16)]
                        cmax = aabb_v[3 + d][pl.ds(o, 16)]
                        ov = ov & (bmin[d] <= cmax) & (cmin <= bmax[d])
                    inc = ov.astype(jnp.int32)
                    pref = plsc.cumsum(inc)
                    pos = pref + (cnt - 1)
                    m = ov & (pos < K)
                    idxv = slot_base + jnp.clip(pos, 0, K - 1)
                    plsc.store_scatter(cand_v, [idxv], jv, mask=m)
                    return (ch + 1, cnt + jnp.sum(inc))

                lax.while_loop(cond, wbody, (jnp.int32(0), jnp.int32(0)))
            return carry
        lax.fori_loop(0, RPT, row_broad, 0)

        def pair_chunk(c, carry):
            p = c * 16 + lanes
            rloc = lax.shift_right_logical(p, 3)
            ig = row_base + rloc
            jj = cand_v[pl.ds(c * 16, 16)]
            valid = jj >= 0
            jc = jnp.maximum(jj, 0)
            t1 = [plsc.load_gather(tri_v[cc], [ig]) for cc in range(9)]
            t2 = [plsc.load_gather(tri_v[cc], [jc]) for cc in range(9)]
            ok = valid & _tri_tri_hit(t1, t2)
            zero = _splat_i32(0)
            plsc.store_scatter(out_v, [p, zero], ig, mask=ok)
            plsc.store_scatter(out_v, [p, zero + 1], jj, mask=ok)
            return carry
        lax.fori_loop(0, NBCH, pair_chunk, 0)

        pstart = row_base * K

        @pl.when(wid < NTILE - 1)
        def _():
            pltpu.sync_copy(out_v.at[pl.ds(0, RPT * K)],
                            out3_hbm.at[b, pl.ds(pstart, RPT * K)])

        @pl.when(wid == NTILE - 1)
        def _():
            pltpu.sync_copy(out_v.at[pl.ds(0, LASTR * K)],
                            out3_hbm.at[b, pl.ds(pstart, LASTR * K)])


@jax.jit
def _bvh_sc(tri_soa):
    mesh = plsc.VectorSubcoreMesh(core_axis_name="c", subcore_axis_name="s")
    fn = functools.partial(
        pl.kernel,
        out_type=jax.ShapeDtypeStruct((B, F * K, 2), jnp.int32),
        mesh=mesh,
        compiler_params=pltpu.CompilerParams(
            use_tc_tiling_on_sc=False, needs_layout_passes=False),
        scratch_types=(
            [pltpu.VMEM((FP,), jnp.float32) for _ in range(9)]
            + [pltpu.VMEM((FP,), jnp.float32) for _ in range(6)]
            + [pltpu.VMEM((CANDW,), jnp.int32),
               pltpu.VMEM((OUTW // 2, 2), jnp.int32)]
        ),
    )(_bvh_body)
    return fn(tri_soa)


def kernel(triangles):
    tri_t = jnp.transpose(triangles, (0, 2, 3, 1)).reshape(B, 9, F)
    tri_p = jnp.pad(tri_t, ((0, 0), (0, 0), (0, FP - F)),
                    constant_values=PAD_VAL)
    return _bvh_sc(tri_p.reshape(-1))


# speedup vs baseline: 165.8221x; 1.1094x over previous
import functools

import jax
import jax.numpy as jnp
from jax import lax
from jax.experimental import pallas as pl
from jax.experimental.pallas import tpu as pltpu
from jax.experimental.pallas import tpu_sc as plsc

F = 5000
B = 2
K = 8
FP = 5120
NCH = FP // 16
NSUB = 16
RPT = 314
NPAIR = RPT * K
NBCH = NPAIR // 16
LASTR = F - (NSUB - 1) * RPT
PAD_VAL = 1e30
EPS = 1e-8


def _cross(ax, ay, az, bx, by, bz):
    return (ay * bz - az * by, az * bx - ax * bz, ax * by - ay * bx)


def _dot3(ax, ay, az, bx, by, bz):
    return ax * bx + ay * by + az * bz


def _interval(d0, d1, d2, p0, p1, p2):
    lo = jnp.full((16,), jnp.inf, jnp.float32)
    hi = jnp.full((16,), -jnp.inf, jnp.float32)
    d = (d0, d1, d2)
    p = (p0, p1, p2)
    for a, b in ((0, 1), (1, 2), (2, 0)):
        da, db = d[a], d[b]
        crossing = (da * db) < 0.0
        denom = da - db
        safe = jnp.where(jnp.abs(denom) > 1e-30, denom, 1.0)
        t = da / safe
        s = p[a] + t * (p[b] - p[a])
        lo = jnp.where(crossing, jnp.minimum(lo, s), lo)
        hi = jnp.where(crossing, jnp.maximum(hi, s), hi)
    return lo, hi


def _tri_tri_hit(v, u):
    v0 = v[0:3]; v1 = v[3:6]; v2 = v[6:9]
    u0 = u[0:3]; u1 = u[3:6]; u2 = u[6:9]
    e1 = [u1[c] - u0[c] for c in range(3)]
    e2 = [u2[c] - u0[c] for c in range(3)]
    n2 = _cross(*e1, *e2)
    d2 = -_dot3(*n2, *u0)
    dv0 = _dot3(*n2, *v0) + d2
    dv1 = _dot3(*n2, *v1) + d2
    dv2 = _dot3(*n2, *v2) + d2
    f1 = [v1[c] - v0[c] for c in range(3)]
    f2 = [v2[c] - v0[c] for c in range(3)]
    n1 = _cross(*f1, *f2)
    d1 = -_dot3(*n1, *v0)
    du0 = _dot3(*n1, *u0) + d1
    du1 = _dot3(*n1, *u1) + d1
    du2 = _dot3(*n1, *u2) + d1
    sep_v = ((dv0 > EPS) & (dv1 > EPS) & (dv2 > EPS)) | \
            ((dv0 < -EPS) & (dv1 < -EPS) & (dv2 < -EPS))
    sep_u = ((du0 > EPS) & (du1 > EPS) & (du2 > EPS)) | \
            ((du0 < -EPS) & (du1 < -EPS) & (du2 < -EPS))
    dd = _cross(*n1, *n2)
    coplanar = _dot3(*dd, *dd) <= EPS
    pv0 = _dot3(*dd, *v0); pv1 = _dot3(*dd, *v1); pv2 = _dot3(*dd, *v2)
    pu0 = _dot3(*dd, *u0); pu1 = _dot3(*dd, *u1); pu2 = _dot3(*dd, *u2)
    lo1, hi1 = _interval(dv0, dv1, dv2, pv0, pv1, pv2)
    lo2, hi2 = _interval(du0, du1, du2, pu0, pu1, pu2)
    seg = jnp.maximum(lo1, lo2) <= jnp.minimum(hi1, hi2)
    return (~sep_v) & (~sep_u) & (~coplanar) & seg


def _splat_i32(x):
    return jnp.full((16,), x, jnp.int32)


def _bvh_body(tri_hbm, out3_hbm, *scratch):
    tri_v = scratch[0:9]
    aabb_v = scratch[9:15]
    cand_v = scratch[15]
    out_v = scratch[16]
    cid = lax.axis_index("c")
    sid = lax.axis_index("s")
    row_base = sid * RPT
    lanes = lax.broadcasted_iota(jnp.int32, (16,), 0)

    for cc in range(9):
        pltpu.sync_copy(tri_hbm.at[pl.ds((cid * 9 + cc) * FP, FP)], tri_v[cc])

    def aabb_body(c, carry):
        o = c * 16
        for d in range(3):
            a0 = tri_v[d][pl.ds(o, 16)]
            a1 = tri_v[3 + d][pl.ds(o, 16)]
            a2 = tri_v[6 + d][pl.ds(o, 16)]
            aabb_v[d][pl.ds(o, 16)] = jnp.minimum(jnp.minimum(a0, a1), a2)
            aabb_v[3 + d][pl.ds(o, 16)] = jnp.maximum(jnp.maximum(a0, a1), a2)
        return carry
    lax.fori_loop(0, NCH, aabb_body, 0)

    def init_cand(c, carry):
        cand_v[pl.ds(c * 16, 16)] = _splat_i32(-1)
        return carry
    lax.fori_loop(0, NBCH, init_cand, 0)

    def init_out(c, carry):
        rows = c * 8 + lax.shift_right_logical(lanes, 1)
        cols = lanes & 1
        plsc.store_scatter(out_v, [rows, cols], _splat_i32(-1))
        return carry
    lax.fori_loop(0, NPAIR // 8, init_out, 0)

    def row_broad(r, carry):
        i = row_base + r

        @pl.when(i < F)
        def _():
            iv = jnp.full((16,), i, jnp.int32)
            bmin = [plsc.load_gather(aabb_v[d], [iv]) for d in range(3)]
            bmax = [plsc.load_gather(aabb_v[3 + d], [iv]) for d in range(3)]
            slot_base = r * K

            def cond(st):
                ch, cnt = st
                return (cnt < K) & (ch < NCH)

            def wbody(st):
                ch, cnt = st
                o = ch * 16
                jv = o + lanes
                ov = jv != i
                for d in range(3):
                    cmin = aabb_v[d][pl.ds(o, 16)]
                    cmax = aabb_v[3 + d][pl.ds(o, 16)]
                    ov = ov & (bmin[d] <= cmax) & (cmin <= bmax[d])
                inc = ov.astype(jnp.int32)
                pref = plsc.cumsum(inc)
                pos = pref + (cnt - 1)
                m = ov & (pos < K)
                idxv = slot_base + jnp.clip(pos, 0, K - 1)
                plsc.store_scatter(cand_v, [idxv], jv, mask=m)
                return (ch + 1, cnt + pref[15])

            lax.while_loop(cond, wbody, (jnp.int32(0), jnp.int32(0)))
        return carry
    lax.fori_loop(0, RPT, row_broad, 0)

    def pair_chunk(c, carry):
        p = c * 16 + lanes
        rloc = lax.shift_right_logical(p, 3)
        ig = row_base + rloc
        jj = cand_v[pl.ds(c * 16, 16)]
        valid = jj >= 0
        jc = jnp.maximum(jj, 0)
        t1 = [plsc.load_gather(tri_v[cc], [ig]) for cc in range(9)]
        t2 = [plsc.load_gather(tri_v[cc], [jc]) for cc in range(9)]
        ok = valid & _tri_tri_hit(t1, t2)
        zero = _splat_i32(0)
        plsc.store_scatter(out_v, [p, zero], ig, mask=ok)
        plsc.store_scatter(out_v, [p, zero + 1], jj, mask=ok)
        return carry
    lax.fori_loop(0, NBCH, pair_chunk, 0)

    pstart = row_base * K

    @pl.when(sid < NSUB - 1)
    def _():
        pltpu.sync_copy(out_v.at[pl.ds(0, RPT * K)],
                        out3_hbm.at[cid, pl.ds(pstart, RPT * K)])

    @pl.when(sid == NSUB - 1)
    def _():
        pltpu.sync_copy(out_v.at[pl.ds(0, LASTR * K)],
                        out3_hbm.at[cid, pl.ds(pstart, LASTR * K)])


@jax.jit
def _bvh_sc(tri_soa):
    mesh = plsc.VectorSubcoreMesh(core_axis_name="c", subcore_axis_name="s")
    fn = functools.partial(
        pl.kernel,
        out_type=jax.ShapeDtypeStruct((B, F * K, 2), jnp.int32),
        mesh=mesh,
        compiler_params=pltpu.CompilerParams(
            use_tc_tiling_on_sc=False, needs_layout_passes=False),
        scratch_types=(
            [pltpu.VMEM((FP,), jnp.float32) for _ in range(9)]
            + [pltpu.VMEM((FP,), jnp.float32) for _ in range(6)]
            + [pltpu.VMEM((NPAIR,), jnp.int32),
               pltpu.VMEM((NPAIR, 2), jnp.int32)]
        ),
    )(_bvh_body)
    return fn(tri_soa)


def kernel(triangles):
    tri_t = jnp.transpose(triangles, (0, 2, 3, 1)).reshape(B, 9, F)
    tri_p = jnp.pad(tri_t, ((0, 0), (0, 0), (0, FP - F)),
                    constant_values=PAD_VAL)
    return _bvh_sc(tri_p.reshape(-1))


# speedup vs baseline: 169.1238x; 1.0199x over previous
import functools

import jax
import jax.numpy as jnp
from jax import lax
from jax.experimental import pallas as pl
from jax.experimental.pallas import tpu as pltpu
from jax.experimental.pallas import tpu_sc as plsc

F = 5000
B = 2
K = 8
FP = 5120
NCH = FP // 16
NSUB = 16
RPT = 314
NPAIR = RPT * K
NBCH = NPAIR // 16
LASTR = F - (NSUB - 1) * RPT
PAD_VAL = 1e30
EPS = 1e-8


def _cross(ax, ay, az, bx, by, bz):
    return (ay * bz - az * by, az * bx - ax * bz, ax * by - ay * bx)


def _dot3(ax, ay, az, bx, by, bz):
    return ax * bx + ay * by + az * bz


def _interval(d0, d1, d2, p0, p1, p2):
    lo = jnp.full((16,), jnp.inf, jnp.float32)
    hi = jnp.full((16,), -jnp.inf, jnp.float32)
    d = (d0, d1, d2)
    p = (p0, p1, p2)
    for a, b in ((0, 1), (1, 2), (2, 0)):
        da, db = d[a], d[b]
        crossing = (da * db) < 0.0
        denom = da - db
        safe = jnp.where(jnp.abs(denom) > 1e-30, denom, 1.0)
        t = da / safe
        s = p[a] + t * (p[b] - p[a])
        lo = jnp.where(crossing, jnp.minimum(lo, s), lo)
        hi = jnp.where(crossing, jnp.maximum(hi, s), hi)
    return lo, hi


def _tri_tri_hit(v, u):
    v0 = v[0:3]; v1 = v[3:6]; v2 = v[6:9]
    u0 = u[0:3]; u1 = u[3:6]; u2 = u[6:9]
    e1 = [u1[c] - u0[c] for c in range(3)]
    e2 = [u2[c] - u0[c] for c in range(3)]
    n2 = _cross(*e1, *e2)
    d2 = -_dot3(*n2, *u0)
    dv0 = _dot3(*n2, *v0) + d2
    dv1 = _dot3(*n2, *v1) + d2
    dv2 = _dot3(*n2, *v2) + d2
    f1 = [v1[c] - v0[c] for c in range(3)]
    f2 = [v2[c] - v0[c] for c in range(3)]
    n1 = _cross(*f1, *f2)
    d1 = -_dot3(*n1, *v0)
    du0 = _dot3(*n1, *u0) + d1
    du1 = _dot3(*n1, *u1) + d1
    du2 = _dot3(*n1, *u2) + d1
    sep_v = ((dv0 > EPS) & (dv1 > EPS) & (dv2 > EPS)) | \
            ((dv0 < -EPS) & (dv1 < -EPS) & (dv2 < -EPS))
    sep_u = ((du0 > EPS) & (du1 > EPS) & (du2 > EPS)) | \
            ((du0 < -EPS) & (du1 < -EPS) & (du2 < -EPS))
    dd = _cross(*n1, *n2)
    coplanar = _dot3(*dd, *dd) <= EPS
    pv0 = _dot3(*dd, *v0); pv1 = _dot3(*dd, *v1); pv2 = _dot3(*dd, *v2)
    pu0 = _dot3(*dd, *u0); pu1 = _dot3(*dd, *u1); pu2 = _dot3(*dd, *u2)
    lo1, hi1 = _interval(dv0, dv1, dv2, pv0, pv1, pv2)
    lo2, hi2 = _interval(du0, du1, du2, pu0, pu1, pu2)
    seg = jnp.maximum(lo1, lo2) <= jnp.minimum(hi1, hi2)
    return (~sep_v) & (~sep_u) & (~coplanar) & seg


def _splat_i32(x):
    return jnp.full((16,), x, jnp.int32)


def _bvh_body(tri_hbm, out3_hbm, *scratch):
    tri_v = scratch[0:9]
    aabb_v = scratch[9:15]
    cand_v = scratch[15]
    out_v = scratch[16]
    cid = lax.axis_index("c")
    sid = lax.axis_index("s")
    row_base = sid * RPT
    lanes = lax.broadcasted_iota(jnp.int32, (16,), 0)

    for cc in range(9):
        pltpu.sync_copy(tri_hbm.at[pl.ds((cid * 9 + cc) * FP, FP)], tri_v[cc])

    def aabb_body(c, carry):
        o = c * 16
        for d in range(3):
            a0 = tri_v[d][pl.ds(o, 16)]
            a1 = tri_v[3 + d][pl.ds(o, 16)]
            a2 = tri_v[6 + d][pl.ds(o, 16)]
            aabb_v[d][pl.ds(o, 16)] = jnp.minimum(jnp.minimum(a0, a1), a2)
            aabb_v[3 + d][pl.ds(o, 16)] = jnp.maximum(jnp.maximum(a0, a1), a2)
        return carry
    lax.fori_loop(0, NCH, aabb_body, 0)

    def init_cand(c, carry):
        cand_v[pl.ds(c * 16, 16)] = _splat_i32(-1)
        return carry
    lax.fori_loop(0, NBCH, init_cand, 0)

    def init_out(c, carry):
        rows = c * 8 + lax.shift_right_logical(lanes, 1)
        cols = lanes & 1
        plsc.store_scatter(out_v, [rows, cols], _splat_i32(-1))
        return carry
    lax.fori_loop(0, NPAIR // 8, init_out, 0)

    def row_broad(r, carry):
        i = row_base + r

        @pl.when(i < F)
        def _():
            iv = jnp.full((16,), i, jnp.int32)
            bmin = [plsc.load_gather(aabb_v[d], [iv]) for d in range(3)]
            bmax = [plsc.load_gather(aabb_v[3 + d], [iv]) for d in range(3)]
            slot_base = r * K

            def cond(st):
                ch, cnt = st
                return (cnt < K) & (ch < NCH)

            def wbody(st):
                ch, cnt = st
                o = ch * 16
                jv = o + lanes
                ov = jv != i
                for d in range(3):
                    cmin = aabb_v[d][pl.ds(o, 16)]
                    cmax = aabb_v[3 + d][pl.ds(o, 16)]
                    ov = ov & (bmin[d] <= cmax) & (cmin <= bmax[d])
                inc = ov.astype(jnp.int32)
                pref = plsc.cumsum(inc)
                pos = pref + (cnt - 1)
                m = ov & (pos < K)
                idxv = slot_base + jnp.clip(pos, 0, K - 1)
                plsc.store_scatter(cand_v, [idxv], jv, mask=m)
                return (ch + 1, cnt + pref[15])

            lax.while_loop(cond, wbody, (jnp.int32(0), jnp.int32(0)))
        return carry
    lax.fori_loop(0, RPT, row_broad, 0)

    @plsc.parallel_loop(0, NBCH, 1, unroll=2)
    def pair_chunk(c):
        p = c * 16 + lanes
        rloc = lax.shift_right_logical(p, 3)
        ig = row_base + rloc
        jj = cand_v[pl.ds(c * 16, 16)]
        valid = jj >= 0
        jc = jnp.maximum(jj, 0)
        t1 = [plsc.load_gather(tri_v[cc], [ig]) for cc in range(9)]
        t2 = [plsc.load_gather(tri_v[cc], [jc]) for cc in range(9)]
        ok = valid & _tri_tri_hit(t1, t2)
        zero = _splat_i32(0)
        plsc.store_scatter(out_v, [p, zero], ig, mask=ok)
        plsc.store_scatter(out_v, [p, zero + 1], jj, mask=ok)

    pstart = row_base * K

    @pl.when(sid < NSUB - 1)
    def _():
        pltpu.sync_copy(out_v.at[pl.ds(0, RPT * K)],
                        out3_hbm.at[cid, pl.ds(pstart, RPT * K)])

    @pl.when(sid == NSUB - 1)
    def _():
        pltpu.sync_copy(out_v.at[pl.ds(0, LASTR * K)],
                        out3_hbm.at[cid, pl.ds(pstart, LASTR * K)])


@jax.jit
def _bvh_sc(tri_soa):
    mesh = plsc.VectorSubcoreMesh(core_axis_name="c", subcore_axis_name="s")
    fn = functools.partial(
        pl.kernel,
        out_type=jax.ShapeDtypeStruct((B, F * K, 2), jnp.int32),
        mesh=mesh,
        compiler_params=pltpu.CompilerParams(
            use_tc_tiling_on_sc=False, needs_layout_passes=False),
        scratch_types=(
            [pltpu.VMEM((FP,), jnp.float32) for _ in range(9)]
            + [pltpu.VMEM((FP,), jnp.float32) for _ in range(6)]
            + [pltpu.VMEM((NPAIR,), jnp.int32),
               pltpu.VMEM((NPAIR, 2), jnp.int32)]
        ),
    )(_bvh_body)
    return fn(tri_soa)


def kernel(triangles):
    tri_t = jnp.transpose(triangles, (0, 2, 3, 1)).reshape(B, 9, F)
    tri_p = jnp.pad(tri_t, ((0, 0), (0, 0), (0, FP - F)),
                    constant_values=PAD_VAL)
    return _bvh_sc(tri_p.reshape(-1))


# speedup vs baseline: 170.9954x; 1.0111x over previous
import functools

import jax
import jax.numpy as jnp
from jax import lax
from jax.experimental import pallas as pl
from jax.experimental.pallas import tpu as pltpu
from jax.experimental.pallas import tpu_sc as plsc

F = 5000
B = 2
K = 8
FP = 5120
NCH = FP // 16
NSUB = 16
RPT = 314
NPAIR = RPT * K
NBCH = NPAIR // 16
LASTR = F - (NSUB - 1) * RPT
PAD_VAL = 1e30
EPS = 1e-8


def _cross(ax, ay, az, bx, by, bz):
    return (ay * bz - az * by, az * bx - ax * bz, ax * by - ay * bx)


def _dot3(ax, ay, az, bx, by, bz):
    return ax * bx + ay * by + az * bz


def _interval(d0, d1, d2, p0, p1, p2):
    lo = jnp.full((16,), jnp.inf, jnp.float32)
    hi = jnp.full((16,), -jnp.inf, jnp.float32)
    d = (d0, d1, d2)
    p = (p0, p1, p2)
    for a, b in ((0, 1), (1, 2), (2, 0)):
        da, db = d[a], d[b]
        crossing = (da * db) < 0.0
        denom = da - db
        safe = jnp.where(jnp.abs(denom) > 1e-30, denom, 1.0)
        t = da / safe
        s = p[a] + t * (p[b] - p[a])
        lo = jnp.where(crossing, jnp.minimum(lo, s), lo)
        hi = jnp.where(crossing, jnp.maximum(hi, s), hi)
    return lo, hi


def _tri_tri_hit(v, u):
    v0 = v[0:3]; v1 = v[3:6]; v2 = v[6:9]
    u0 = u[0:3]; u1 = u[3:6]; u2 = u[6:9]
    e1 = [u1[c] - u0[c] for c in range(3)]
    e2 = [u2[c] - u0[c] for c in range(3)]
    n2 = _cross(*e1, *e2)
    d2 = -_dot3(*n2, *u0)
    dv0 = _dot3(*n2, *v0) + d2
    dv1 = _dot3(*n2, *v1) + d2
    dv2 = _dot3(*n2, *v2) + d2
    f1 = [v1[c] - v0[c] for c in range(3)]
    f2 = [v2[c] - v0[c] for c in range(3)]
    n1 = _cross(*f1, *f2)
    d1 = -_dot3(*n1, *v0)
    du0 = _dot3(*n1, *u0) + d1
    du1 = _dot3(*n1, *u1) + d1
    du2 = _dot3(*n1, *u2) + d1
    sep_v = ((dv0 > EPS) & (dv1 > EPS) & (dv2 > EPS)) | \
            ((dv0 < -EPS) & (dv1 < -EPS) & (dv2 < -EPS))
    sep_u = ((du0 > EPS) & (du1 > EPS) & (du2 > EPS)) | \
            ((du0 < -EPS) & (du1 < -EPS) & (du2 < -EPS))
    dd = _cross(*n1, *n2)
    coplanar = _dot3(*dd, *dd) <= EPS
    pv0 = _dot3(*dd, *v0); pv1 = _dot3(*dd, *v1); pv2 = _dot3(*dd, *v2)
    pu0 = _dot3(*dd, *u0); pu1 = _dot3(*dd, *u1); pu2 = _dot3(*dd, *u2)
    lo1, hi1 = _interval(dv0, dv1, dv2, pv0, pv1, pv2)
    lo2, hi2 = _interval(du0, du1, du2, pu0, pu1, pu2)
    seg = jnp.maximum(lo1, lo2) <= jnp.minimum(hi1, hi2)
    return (~sep_v) & (~sep_u) & (~coplanar) & seg


def _splat_i32(x):
    return jnp.full((16,), x, jnp.int32)


def _bvh_body(tri_hbm, out3_hbm, *scratch):
    tri_v = scratch[0:9]
    aabb_v = scratch[9:15]
    cand_v = scratch[15]
    out_v = scratch[16]
    cid = lax.axis_index("c")
    sid = lax.axis_index("s")
    row_base = sid * RPT
    lanes = lax.broadcasted_iota(jnp.int32, (16,), 0)

    for cc in range(9):
        pltpu.sync_copy(tri_hbm.at[pl.ds((cid * 9 + cc) * FP, FP)], tri_v[cc])

    def aabb_body(c, carry):
        o = c * 16
        for d in range(3):
            a0 = tri_v[d][pl.ds(o, 16)]
            a1 = tri_v[3 + d][pl.ds(o, 16)]
            a2 = tri_v[6 + d][pl.ds(o, 16)]
            aabb_v[d][pl.ds(o, 16)] = jnp.minimum(jnp.minimum(a0, a1), a2)
            aabb_v[3 + d][pl.ds(o, 16)] = jnp.maximum(jnp.maximum(a0, a1), a2)
        return carry
    lax.fori_loop(0, NCH, aabb_body, 0)

    def init_cand(c, carry):
        cand_v[pl.ds(c * 16, 16)] = _splat_i32(-1)
        return carry
    lax.fori_loop(0, NBCH, init_cand, 0)

    def init_out(c, carry):
        rows = c * 8 + lax.shift_right_logical(lanes, 1)
        cols = lanes & 1
        plsc.store_scatter(out_v, [rows, cols], _splat_i32(-1))
        return carry
    lax.fori_loop(0, NPAIR // 8, init_out, 0)

    @plsc.parallel_loop(0, RPT, 1, unroll=2)
    def row_broad(r):
        i = row_base + r

        @pl.when(i < F)
        def _():
            iv = jnp.full((16,), i, jnp.int32)
            bmin = [plsc.load_gather(aabb_v[d], [iv]) for d in range(3)]
            bmax = [plsc.load_gather(aabb_v[3 + d], [iv]) for d in range(3)]
            slot_base = r * K

            def cond(st):
                ch, cnt = st
                return (cnt < K) & (ch < NCH)

            def wbody(st):
                ch, cnt = st
                o = ch * 16
                jv = o + lanes
                ov = jv != i
                for d in range(3):
                    cmin = aabb_v[d][pl.ds(o, 16)]
                    cmax = aabb_v[3 + d][pl.ds(o, 16)]
                    ov = ov & (bmin[d] <= cmax) & (cmin <= bmax[d])
                inc = ov.astype(jnp.int32)
                pref = plsc.cumsum(inc)
                pos = pref + (cnt - 1)
                m = ov & (pos < K)
                idxv = slot_base + jnp.clip(pos, 0, K - 1)
                plsc.store_scatter(cand_v, [idxv], jv, mask=m)
                return (ch + 1, cnt + pref[15])

            lax.while_loop(cond, wbody, (jnp.int32(0), jnp.int32(0)))

    @plsc.parallel_loop(0, NBCH, 1, unroll=2)
    def pair_chunk(c):
        p = c * 16 + lanes
        rloc = lax.shift_right_logical(p, 3)
        ig = row_base + rloc
        jj = cand_v[pl.ds(c * 16, 16)]
        valid = jj >= 0
        jc = jnp.maximum(jj, 0)
        t1 = [plsc.load_gather(tri_v[cc], [ig]) for cc in range(9)]
        t2 = [plsc.load_gather(tri_v[cc], [jc]) for cc in range(9)]
        ok = valid & _tri_tri_hit(t1, t2)
        zero = _splat_i32(0)
        plsc.store_scatter(out_v, [p, zero], ig, mask=ok)
        plsc.store_scatter(out_v, [p, zero + 1], jj, mask=ok)

    pstart = row_base * K

    @pl.when(sid < NSUB - 1)
    def _():
        pltpu.sync_copy(out_v.at[pl.ds(0, RPT * K)],
                        out3_hbm.at[cid, pl.ds(pstart, RPT * K)])

    @pl.when(sid == NSUB - 1)
    def _():
        pltpu.sync_copy(out_v.at[pl.ds(0, LASTR * K)],
                        out3_hbm.at[cid, pl.ds(pstart, LASTR * K)])


@jax.jit
def _bvh_sc(tri_soa):
    mesh = plsc.VectorSubcoreMesh(core_axis_name="c", subcore_axis_name="s")
    fn = functools.partial(
        pl.kernel,
        out_type=jax.ShapeDtypeStruct((B, F * K, 2), jnp.int32),
        mesh=mesh,
        compiler_params=pltpu.CompilerParams(
            use_tc_tiling_on_sc=False, needs_layout_passes=False),
        scratch_types=(
            [pltpu.VMEM((FP,), jnp.float32) for _ in range(9)]
            + [pltpu.VMEM((FP,), jnp.float32) for _ in range(6)]
            + [pltpu.VMEM((NPAIR,), jnp.int32),
               pltpu.VMEM((NPAIR, 2), jnp.int32)]
        ),
    )(_bvh_body)
    return fn(tri_soa)


def kernel(triangles):
    tri_t = jnp.transpose(triangles, (0, 2, 3, 1)).reshape(B, 9, F)
    tri_p = jnp.pad(tri_t, ((0, 0), (0, 0), (0, FP - F)),
                    constant_values=PAD_VAL)
    return _bvh_sc(tri_p.reshape(-1))


# speedup vs baseline: 311.5056x; 1.8217x over previous
import functools

import jax
import jax.numpy as jnp
from jax import lax
from jax.experimental import pallas as pl
from jax.experimental.pallas import tpu as pltpu
from jax.experimental.pallas import tpu_sc as plsc

F = 5000
B = 2
K = 8
FP = 5120
NCH = FP // 16
NSUB = 16
RPT = 314
NPAIR = RPT * K
NBCH = NPAIR // 16
LASTR = F - (NSUB - 1) * RPT
PAD_VAL = 1e30
EPS = 1e-8


def _cross(ax, ay, az, bx, by, bz):
    return (ay * bz - az * by, az * bx - ax * bz, ax * by - ay * bx)


def _dot3(ax, ay, az, bx, by, bz):
    return ax * bx + ay * by + az * bz


def _interval(d0, d1, d2, p0, p1, p2):
    lo = jnp.full((16,), jnp.inf, jnp.float32)
    hi = jnp.full((16,), -jnp.inf, jnp.float32)
    d = (d0, d1, d2)
    p = (p0, p1, p2)
    for a, b in ((0, 1), (1, 2), (2, 0)):
        da, db = d[a], d[b]
        crossing = (da * db) < 0.0
        denom = da - db
        safe = jnp.where(jnp.abs(denom) > 1e-30, denom, 1.0)
        t = da / safe
        s = p[a] + t * (p[b] - p[a])
        lo = jnp.where(crossing, jnp.minimum(lo, s), lo)
        hi = jnp.where(crossing, jnp.maximum(hi, s), hi)
    return lo, hi


def _tri_tri_hit(v, u):
    v0 = v[0:3]; v1 = v[3:6]; v2 = v[6:9]
    u0 = u[0:3]; u1 = u[3:6]; u2 = u[6:9]
    e1 = [u1[c] - u0[c] for c in range(3)]
    e2 = [u2[c] - u0[c] for c in range(3)]
    n2 = _cross(*e1, *e2)
    d2 = -_dot3(*n2, *u0)
    dv0 = _dot3(*n2, *v0) + d2
    dv1 = _dot3(*n2, *v1) + d2
    dv2 = _dot3(*n2, *v2) + d2
    f1 = [v1[c] - v0[c] for c in range(3)]
    f2 = [v2[c] - v0[c] for c in range(3)]
    n1 = _cross(*f1, *f2)
    d1 = -_dot3(*n1, *v0)
    du0 = _dot3(*n1, *u0) + d1
    du1 = _dot3(*n1, *u1) + d1
    du2 = _dot3(*n1, *u2) + d1
    sep_v = ((dv0 > EPS) & (dv1 > EPS) & (dv2 > EPS)) | \
            ((dv0 < -EPS) & (dv1 < -EPS) & (dv2 < -EPS))
    sep_u = ((du0 > EPS) & (du1 > EPS) & (du2 > EPS)) | \
            ((du0 < -EPS) & (du1 < -EPS) & (du2 < -EPS))
    dd = _cross(*n1, *n2)
    coplanar = _dot3(*dd, *dd) <= EPS
    pv0 = _dot3(*dd, *v0); pv1 = _dot3(*dd, *v1); pv2 = _dot3(*dd, *v2)
    pu0 = _dot3(*dd, *u0); pu1 = _dot3(*dd, *u1); pu2 = _dot3(*dd, *u2)
    lo1, hi1 = _interval(dv0, dv1, dv2, pv0, pv1, pv2)
    lo2, hi2 = _interval(du0, du1, du2, pu0, pu1, pu2)
    seg = jnp.maximum(lo1, lo2) <= jnp.minimum(hi1, hi2)
    return (~sep_v) & (~sep_u) & (~coplanar) & seg


def _splat_i32(x):
    return jnp.full((16,), x, jnp.int32)


def _bvh_body(tri_hbm, oi_hbm, oj_hbm, *scratch):
    tri_v = scratch[0:9]
    aabb_v = scratch[9:15]
    cand_v = scratch[15]
    oi_v = scratch[16]
    oj_v = scratch[17]
    cid = lax.axis_index("c")
    sid = lax.axis_index("s")
    row_base = sid * RPT
    lanes = lax.broadcasted_iota(jnp.int32, (16,), 0)

    for cc in range(9):
        pltpu.sync_copy(tri_hbm.at[pl.ds((cid * 9 + cc) * FP, FP)], tri_v[cc])

    def aabb_body(c, carry):
        o = c * 16
        for d in range(3):
            a0 = tri_v[d][pl.ds(o, 16)]
            a1 = tri_v[3 + d][pl.ds(o, 16)]
            a2 = tri_v[6 + d][pl.ds(o, 16)]
            aabb_v[d][pl.ds(o, 16)] = jnp.minimum(jnp.minimum(a0, a1), a2)
            aabb_v[3 + d][pl.ds(o, 16)] = jnp.maximum(jnp.maximum(a0, a1), a2)
        return carry
    lax.fori_loop(0, NCH, aabb_body, 0)

    def init_cand(c, carry):
        cand_v[pl.ds(c * 16, 16)] = _splat_i32(-1)
        return carry
    lax.fori_loop(0, NBCH, init_cand, 0)

    def init_out(c, carry):
        oi_v[pl.ds(c * 16, 16)] = _splat_i32(-1)
        oj_v[pl.ds(c * 16, 16)] = _splat_i32(-1)
        return carry
    lax.fori_loop(0, NBCH, init_out, 0)

    @plsc.parallel_loop(0, RPT, 1, unroll=2)
    def row_broad(r):
        i = row_base + r

        @pl.when(i < F)
        def _():
            iv = jnp.full((16,), i, jnp.int32)
            bmin = [plsc.load_gather(aabb_v[d], [iv]) for d in range(3)]
            bmax = [plsc.load_gather(aabb_v[3 + d], [iv]) for d in range(3)]
            slot_base = r * K

            def cond(st):
                ch, cnt = st
                return (cnt < K) & (ch < NCH)

            def wbody(st):
                ch, cnt = st
                o = ch * 16
                jv = o + lanes
                ov = jv != i
                for d in range(3):
                    cmin = aabb_v[d][pl.ds(o, 16)]
                    cmax = aabb_v[3 + d][pl.ds(o, 16)]
                    ov = ov & (bmin[d] <= cmax) & (cmin <= bmax[d])
                inc = ov.astype(jnp.int32)
                pref = plsc.cumsum(inc)
                pos = pref + (cnt - 1)
                m = ov & (pos < K)
                idxv = slot_base + jnp.clip(pos, 0, K - 1)
                plsc.store_scatter(cand_v, [idxv], jv, mask=m)
                return (ch + 1, cnt + pref[15])

            lax.while_loop(cond, wbody, (jnp.int32(0), jnp.int32(0)))

    @plsc.parallel_loop(0, NBCH, 1, unroll=2)
    def pair_chunk(c):
        p = c * 16 + lanes
        rloc = lax.shift_right_logical(p, 3)
        ig = row_base + rloc
        jj = cand_v[pl.ds(c * 16, 16)]
        valid = jj >= 0
        jc = jnp.maximum(jj, 0)
        t1 = [plsc.load_gather(tri_v[cc], [ig]) for cc in range(9)]
        t2 = [plsc.load_gather(tri_v[cc], [jc]) for cc in range(9)]
        ok = valid & _tri_tri_hit(t1, t2)
        plsc.store_scatter(oi_v, [p], ig, mask=ok)
        plsc.store_scatter(oj_v, [p], jj, mask=ok)

    pstart = row_base * K

    @pl.when(sid < NSUB - 1)
    def _():
        pltpu.sync_copy(oi_v.at[pl.ds(0, RPT * K)],
                        oi_hbm.at[cid, pl.ds(pstart, RPT * K)])
        pltpu.sync_copy(oj_v.at[pl.ds(0, RPT * K)],
                        oj_hbm.at[cid, pl.ds(pstart, RPT * K)])

    @pl.when(sid == NSUB - 1)
    def _():
        pltpu.sync_copy(oi_v.at[pl.ds(0, LASTR * K)],
                        oi_hbm.at[cid, pl.ds(pstart, LASTR * K)])
        pltpu.sync_copy(oj_v.at[pl.ds(0, LASTR * K)],
                        oj_hbm.at[cid, pl.ds(pstart, LASTR * K)])


@jax.jit
def _bvh_sc(tri_soa):
    mesh = plsc.VectorSubcoreMesh(core_axis_name="c", subcore_axis_name="s")
    fn = functools.partial(
        pl.kernel,
        out_type=(jax.ShapeDtypeStruct((B, F * K), jnp.int32),
                  jax.ShapeDtypeStruct((B, F * K), jnp.int32)),
        mesh=mesh,
        compiler_params=pltpu.CompilerParams(
            use_tc_tiling_on_sc=False, needs_layout_passes=False),
        scratch_types=(
            [pltpu.VMEM((FP,), jnp.float32) for _ in range(9)]
            + [pltpu.VMEM((FP,), jnp.float32) for _ in range(6)]
            + [pltpu.VMEM((NPAIR,), jnp.int32),
               pltpu.VMEM((NPAIR,), jnp.int32),
               pltpu.VMEM((NPAIR,), jnp.int32)]
        ),
    )(_bvh_body)
    return fn(tri_soa)


def kernel(triangles):
    tri_t = jnp.transpose(triangles, (0, 2, 3, 1)).reshape(B, 9, F)
    tri_p = jnp.pad(tri_t, ((0, 0), (0, 0), (0, FP - F)),
                    constant_values=PAD_VAL)
    oi, oj = _bvh_sc(tri_p.reshape(-1))
    return jnp.stack([oi, oj], axis=-1)


# speedup vs baseline: 314.3350x; 1.0091x over previous
import functools

import jax
import jax.numpy as jnp
from jax import lax
from jax.experimental import pallas as pl
from jax.experimental.pallas import tpu as pltpu
from jax.experimental.pallas import tpu_sc as plsc

F = 5000
B = 2
K = 8
FP = 5120
NCH = FP // 16
NSUB = 16
RPT = 314
NPAIR = RPT * K
NBCH = NPAIR // 16
LASTR = F - (NSUB - 1) * RPT
PAD_VAL = 1e30
EPS = 1e-8


def _cross(ax, ay, az, bx, by, bz):
    return (ay * bz - az * by, az * bx - ax * bz, ax * by - ay * bx)


def _dot3(ax, ay, az, bx, by, bz):
    return ax * bx + ay * by + az * bz


def _interval(d0, d1, d2, p0, p1, p2):
    lo = jnp.full((16,), jnp.inf, jnp.float32)
    hi = jnp.full((16,), -jnp.inf, jnp.float32)
    d = (d0, d1, d2)
    p = (p0, p1, p2)
    for a, b in ((0, 1), (1, 2), (2, 0)):
        da, db = d[a], d[b]
        crossing = (da * db) < 0.0
        denom = da - db
        safe = jnp.where(jnp.abs(denom) > 1e-30, denom, 1.0)
        t = da / safe
        s = p[a] + t * (p[b] - p[a])
        lo = jnp.where(crossing, jnp.minimum(lo, s), lo)
        hi = jnp.where(crossing, jnp.maximum(hi, s), hi)
    return lo, hi


def _tri_tri_hit(v, u):
    v0 = v[0:3]; v1 = v[3:6]; v2 = v[6:9]
    u0 = u[0:3]; u1 = u[3:6]; u2 = u[6:9]
    e1 = [u1[c] - u0[c] for c in range(3)]
    e2 = [u2[c] - u0[c] for c in range(3)]
    n2 = _cross(*e1, *e2)
    d2 = -_dot3(*n2, *u0)
    dv0 = _dot3(*n2, *v0) + d2
    dv1 = _dot3(*n2, *v1) + d2
    dv2 = _dot3(*n2, *v2) + d2
    f1 = [v1[c] - v0[c] for c in range(3)]
    f2 = [v2[c] - v0[c] for c in range(3)]
    n1 = _cross(*f1, *f2)
    d1 = -_dot3(*n1, *v0)
    du0 = _dot3(*n1, *u0) + d1
    du1 = _dot3(*n1, *u1) + d1
    du2 = _dot3(*n1, *u2) + d1
    sep_v = ((dv0 > EPS) & (dv1 > EPS) & (dv2 > EPS)) | \
            ((dv0 < -EPS) & (dv1 < -EPS) & (dv2 < -EPS))
    sep_u = ((du0 > EPS) & (du1 > EPS) & (du2 > EPS)) | \
            ((du0 < -EPS) & (du1 < -EPS) & (du2 < -EPS))
    dd = _cross(*n1, *n2)
    coplanar = _dot3(*dd, *dd) <= EPS
    pv0 = _dot3(*dd, *v0); pv1 = _dot3(*dd, *v1); pv2 = _dot3(*dd, *v2)
    pu0 = _dot3(*dd, *u0); pu1 = _dot3(*dd, *u1); pu2 = _dot3(*dd, *u2)
    lo1, hi1 = _interval(dv0, dv1, dv2, pv0, pv1, pv2)
    lo2, hi2 = _interval(du0, du1, du2, pu0, pu1, pu2)
    seg = jnp.maximum(lo1, lo2) <= jnp.minimum(hi1, hi2)
    return (~sep_v) & (~sep_u) & (~coplanar) & seg


def _splat_i32(x):
    return jnp.full((16,), x, jnp.int32)


def _bvh_body(tri_hbm, oi_hbm, oj_hbm, *scratch):
    tri_v = scratch[0:9]
    aabb_v = scratch[9:15]
    cand_v = scratch[15]
    oi_v = scratch[16]
    oj_v = scratch[17]
    cnt_v = scratch[18]
    cid = lax.axis_index("c")
    sid = lax.axis_index("s")
    row_base = sid * RPT
    lanes = lax.broadcasted_iota(jnp.int32, (16,), 0)

    for cc in range(9):
        pltpu.sync_copy(tri_hbm.at[pl.ds((cid * 9 + cc) * FP, FP)], tri_v[cc])

    def aabb_body(c, carry):
        o = c * 16
        for d in range(3):
            a0 = tri_v[d][pl.ds(o, 16)]
            a1 = tri_v[3 + d][pl.ds(o, 16)]
            a2 = tri_v[6 + d][pl.ds(o, 16)]
            aabb_v[d][pl.ds(o, 16)] = jnp.minimum(jnp.minimum(a0, a1), a2)
            aabb_v[3 + d][pl.ds(o, 16)] = jnp.maximum(jnp.maximum(a0, a1), a2)
        return carry
    lax.fori_loop(0, NCH, aabb_body, 0)

    def init_cand(c, carry):
        cand_v[pl.ds(c * 16, 16)] = _splat_i32(-1)
        return carry
    lax.fori_loop(0, NBCH, init_cand, 0)

    def init_out(c, carry):
        oi_v[pl.ds(c * 16, 16)] = _splat_i32(-1)
        oj_v[pl.ds(c * 16, 16)] = _splat_i32(-1)
        return carry
    lax.fori_loop(0, NBCH, init_out, 0)

    def init_cnt(c, carry):
        cnt_v[pl.ds(c * 16, 16)] = _splat_i32(K)
        return carry
    lax.fori_loop(0, 20, init_cnt, 0)

    c0min = [aabb_v[d][pl.ds(0, 16)] for d in range(3)]
    c0max = [aabb_v[3 + d][pl.ds(0, 16)] for d in range(3)]

    @plsc.parallel_loop(0, RPT, 1, unroll=4)
    def row_pass1(r):
        i = row_base + r

        @pl.when(i < F)
        def _():
            iv = jnp.full((16,), i, jnp.int32)
            bmin = [plsc.load_gather(aabb_v[d], [iv]) for d in range(3)]
            bmax = [plsc.load_gather(aabb_v[3 + d], [iv]) for d in range(3)]
            ov = lanes != i
            for d in range(3):
                ov = ov & (bmin[d] <= c0max[d]) & (c0min[d] <= bmax[d])
            inc = ov.astype(jnp.int32)
            pref = plsc.cumsum(inc)
            pos = pref - 1
            m = ov & (pos < K)
            idxv = r * K + jnp.clip(pos, 0, K - 1)
            plsc.store_scatter(cand_v, [idxv], lanes, mask=m)
            rv = jnp.full((16,), r, jnp.int32)
            plsc.store_scatter(cnt_v, [rv], pref, mask=lanes == 15)

    def row_pass2(r, carry):
        i = row_base + r
        cnt0 = plsc.load_gather(cnt_v, [jnp.full((16,), r, jnp.int32)])[0]

        @pl.when(cnt0 < K)
        def _():
            iv = jnp.full((16,), i, jnp.int32)
            bmin = [plsc.load_gather(aabb_v[d], [iv]) for d in range(3)]
            bmax = [plsc.load_gather(aabb_v[3 + d], [iv]) for d in range(3)]
            slot_base = r * K

            def cond(st):
                ch, cnt = st
                return (cnt < K) & (ch < NCH)

            def wbody(st):
                ch, cnt = st
                o = ch * 16
                jv = o + lanes
                ov = jv != i
                for d in range(3):
                    cmin = aabb_v[d][pl.ds(o, 16)]
                    cmax = aabb_v[3 + d][pl.ds(o, 16)]
                    ov = ov & (bmin[d] <= cmax) & (cmin <= bmax[d])
                inc = ov.astype(jnp.int32)
                pref = plsc.cumsum(inc)
                pos = pref + (cnt - 1)
                m = ov & (pos < K)
                idxv = slot_base + jnp.clip(pos, 0, K - 1)
                plsc.store_scatter(cand_v, [idxv], jv, mask=m)
                return (ch + 1, cnt + pref[15])

            lax.while_loop(cond, wbody, (jnp.int32(1), cnt0))
        return carry
    lax.fori_loop(0, RPT, row_pass2, 0)

    @plsc.parallel_loop(0, NBCH, 1, unroll=2)
    def pair_chunk(c):
        p = c * 16 + lanes
        rloc = lax.shift_right_logical(p, 3)
        ig = row_base + rloc
        jj = cand_v[pl.ds(c * 16, 16)]
        valid = jj >= 0
        jc = jnp.maximum(jj, 0)
        t1 = [plsc.load_gather(tri_v[cc], [ig]) for cc in range(9)]
        t2 = [plsc.load_gather(tri_v[cc], [jc]) for cc in range(9)]
        ok = valid & _tri_tri_hit(t1, t2)
        plsc.store_scatter(oi_v, [p], ig, mask=ok)
        plsc.store_scatter(oj_v, [p], jj, mask=ok)

    pstart = row_base * K

    @pl.when(sid < NSUB - 1)
    def _():
        pltpu.sync_copy(oi_v.at[pl.ds(0, RPT * K)],
                        oi_hbm.at[cid, pl.ds(pstart, RPT * K)])
        pltpu.sync_copy(oj_v.at[pl.ds(0, RPT * K)],
                        oj_hbm.at[cid, pl.ds(pstart, RPT * K)])

    @pl.when(sid == NSUB - 1)
    def _():
        pltpu.sync_copy(oi_v.at[pl.ds(0, LASTR * K)],
                        oi_hbm.at[cid, pl.ds(pstart, LASTR * K)])
        pltpu.sync_copy(oj_v.at[pl.ds(0, LASTR * K)],
                        oj_hbm.at[cid, pl.ds(pstart, LASTR * K)])


@jax.jit
def _bvh_sc(tri_soa):
    mesh = plsc.VectorSubcoreMesh(core_axis_name="c", subcore_axis_name="s")
    fn = functools.partial(
        pl.kernel,
        out_type=(jax.ShapeDtypeStruct((B, F * K), jnp.int32),
                  jax.ShapeDtypeStruct((B, F * K), jnp.int32)),
        mesh=mesh,
        compiler_params=pltpu.CompilerParams(
            use_tc_tiling_on_sc=False, needs_layout_passes=False),
        scratch_types=(
            [pltpu.VMEM((FP,), jnp.float32) for _ in range(9)]
            + [pltpu.VMEM((FP,), jnp.float32) for _ in range(6)]
            + [pltpu.VMEM((NPAIR,), jnp.int32),
               pltpu.VMEM((NPAIR,), jnp.int32),
               pltpu.VMEM((NPAIR,), jnp.int32),
               pltpu.VMEM((320,), jnp.int32)]
        ),
    )(_bvh_body)
    return fn(tri_soa)


def kernel(triangles):
    tri_t = jnp.transpose(triangles, (0, 2, 3, 1)).reshape(B, 9, F)
    tri_p = jnp.pad(tri_t, ((0, 0), (0, 0), (0, FP - F)),
                    constant_values=PAD_VAL)
    oi, oj = _bvh_sc(tri_p.reshape(-1))
    return jnp.stack([oi, oj], axis=-1)


# speedup vs baseline: 332.7841x; 1.0587x over previous
import functools

import jax
import jax.numpy as jnp
from jax import lax
from jax.experimental import pallas as pl
from jax.experimental.pallas import tpu as pltpu
from jax.experimental.pallas import tpu_sc as plsc

F = 5000
B = 2
K = 8
FP = 5120
NCH = FP // 16
NSUB = 16
RPT = 314
NPAIR = RPT * K
NBCH = NPAIR // 16
LASTR = F - (NSUB - 1) * RPT
PAD_VAL = 1e30
EPS = 1e-8


def _cross(ax, ay, az, bx, by, bz):
    return (ay * bz - az * by, az * bx - ax * bz, ax * by - ay * bx)


def _dot3(ax, ay, az, bx, by, bz):
    return ax * bx + ay * by + az * bz


def _interval(d0, d1, d2, p0, p1, p2):
    lo = jnp.full((16,), jnp.inf, jnp.float32)
    hi = jnp.full((16,), -jnp.inf, jnp.float32)
    d = (d0, d1, d2)
    p = (p0, p1, p2)
    for a, b in ((0, 1), (1, 2), (2, 0)):
        da, db = d[a], d[b]
        crossing = (da * db) < 0.0
        denom = da - db
        safe = jnp.where(jnp.abs(denom) > 1e-30, denom, 1.0)
        t = da / safe
        s = p[a] + t * (p[b] - p[a])
        lo = jnp.where(crossing, jnp.minimum(lo, s), lo)
        hi = jnp.where(crossing, jnp.maximum(hi, s), hi)
    return lo, hi


def _tri_tri_hit(v, u):
    v0 = v[0:3]; v1 = v[3:6]; v2 = v[6:9]
    u0 = u[0:3]; u1 = u[3:6]; u2 = u[6:9]
    e1 = [u1[c] - u0[c] for c in range(3)]
    e2 = [u2[c] - u0[c] for c in range(3)]
    n2 = _cross(*e1, *e2)
    d2 = -_dot3(*n2, *u0)
    dv0 = _dot3(*n2, *v0) + d2
    dv1 = _dot3(*n2, *v1) + d2
    dv2 = _dot3(*n2, *v2) + d2
    f1 = [v1[c] - v0[c] for c in range(3)]
    f2 = [v2[c] - v0[c] for c in range(3)]
    n1 = _cross(*f1, *f2)
    d1 = -_dot3(*n1, *v0)
    du0 = _dot3(*n1, *u0) + d1
    du1 = _dot3(*n1, *u1) + d1
    du2 = _dot3(*n1, *u2) + d1
    sep_v = ((dv0 > EPS) & (dv1 > EPS) & (dv2 > EPS)) | \
            ((dv0 < -EPS) & (dv1 < -EPS) & (dv2 < -EPS))
    sep_u = ((du0 > EPS) & (du1 > EPS) & (du2 > EPS)) | \
            ((du0 < -EPS) & (du1 < -EPS) & (du2 < -EPS))
    dd = _cross(*n1, *n2)
    coplanar = _dot3(*dd, *dd) <= EPS
    pv0 = _dot3(*dd, *v0); pv1 = _dot3(*dd, *v1); pv2 = _dot3(*dd, *v2)
    pu0 = _dot3(*dd, *u0); pu1 = _dot3(*dd, *u1); pu2 = _dot3(*dd, *u2)
    lo1, hi1 = _interval(dv0, dv1, dv2, pv0, pv1, pv2)
    lo2, hi2 = _interval(du0, du1, du2, pu0, pu1, pu2)
    seg = jnp.maximum(lo1, lo2) <= jnp.minimum(hi1, hi2)
    return (~sep_v) & (~sep_u) & (~coplanar) & seg


def _splat_i32(x):
    return jnp.full((16,), x, jnp.int32)


def _bvh_body(tri_hbm, oi_hbm, oj_hbm, *scratch):
    tri_v = scratch[0:9]
    aabb_v = scratch[9:15]
    cand_v = scratch[15]
    oi_v = scratch[16]
    oj_v = scratch[17]
    cnt_v = scratch[18]
    cid = lax.axis_index("c")
    sid = lax.axis_index("s")
    row_base = sid * RPT
    lanes = lax.broadcasted_iota(jnp.int32, (16,), 0)

    dma_sem = scratch[19]
    handles = [
        pltpu.make_async_copy(tri_hbm.at[pl.ds((cid * 9 + cc) * FP, FP)],
                              tri_v[cc], dma_sem)
        for cc in range(9)
    ]
    for h in handles:
        h.start()
    for h in handles:
        h.wait()

    @plsc.parallel_loop(0, NCH, 1, unroll=4)
    def aabb_body(c):
        o = c * 16
        for d in range(3):
            a0 = tri_v[d][pl.ds(o, 16)]
            a1 = tri_v[3 + d][pl.ds(o, 16)]
            a2 = tri_v[6 + d][pl.ds(o, 16)]
            aabb_v[d][pl.ds(o, 16)] = jnp.minimum(jnp.minimum(a0, a1), a2)
            aabb_v[3 + d][pl.ds(o, 16)] = jnp.maximum(jnp.maximum(a0, a1), a2)

    def init_cand(c, carry):
        cand_v[pl.ds(c * 16, 16)] = _splat_i32(-1)
        return carry
    lax.fori_loop(0, NBCH, init_cand, 0)

    def init_out(c, carry):
        oi_v[pl.ds(c * 16, 16)] = _splat_i32(-1)
        oj_v[pl.ds(c * 16, 16)] = _splat_i32(-1)
        return carry
    lax.fori_loop(0, NBCH, init_out, 0)

    def init_cnt(c, carry):
        cnt_v[pl.ds(c * 16, 16)] = _splat_i32(K)
        return carry
    lax.fori_loop(0, 20, init_cnt, 0)

    c0min = [aabb_v[d][pl.ds(0, 16)] for d in range(3)]
    c0max = [aabb_v[3 + d][pl.ds(0, 16)] for d in range(3)]

    @plsc.parallel_loop(0, RPT, 1, unroll=4)
    def row_pass1(r):
        i = row_base + r

        @pl.when(i < F)
        def _():
            iv = jnp.full((16,), i, jnp.int32)
            bmin = [plsc.load_gather(aabb_v[d], [iv]) for d in range(3)]
            bmax = [plsc.load_gather(aabb_v[3 + d], [iv]) for d in range(3)]
            ov = lanes != i
            for d in range(3):
                ov = ov & (bmin[d] <= c0max[d]) & (c0min[d] <= bmax[d])
            inc = ov.astype(jnp.int32)
            pref = plsc.cumsum(inc)
            pos = pref - 1
            m = ov & (pos < K)
            idxv = r * K + jnp.clip(pos, 0, K - 1)
            plsc.store_scatter(cand_v, [idxv], lanes, mask=m)
            rv = jnp.full((16,), r, jnp.int32)
            plsc.store_scatter(cnt_v, [rv], pref, mask=lanes == 15)

    def row_pass2(r, carry):
        i = row_base + r
        cnt0 = plsc.load_gather(cnt_v, [jnp.full((16,), r, jnp.int32)])[0]

        @pl.when(cnt0 < K)
        def _():
            iv = jnp.full((16,), i, jnp.int32)
            bmin = [plsc.load_gather(aabb_v[d], [iv]) for d in range(3)]
            bmax = [plsc.load_gather(aabb_v[3 + d], [iv]) for d in range(3)]
            slot_base = r * K

            def cond(st):
                ch, cnt = st
                return (cnt < K) & (ch < NCH)

            def wbody(st):
                ch, cnt = st
                o = ch * 16
                jv = o + lanes
                ov = jv != i
                for d in range(3):
                    cmin = aabb_v[d][pl.ds(o, 16)]
                    cmax = aabb_v[3 + d][pl.ds(o, 16)]
                    ov = ov & (bmin[d] <= cmax) & (cmin <= bmax[d])
                inc = ov.astype(jnp.int32)
                pref = plsc.cumsum(inc)
                pos = pref + (cnt - 1)
                m = ov & (pos < K)
                idxv = slot_base + jnp.clip(pos, 0, K - 1)
                plsc.store_scatter(cand_v, [idxv], jv, mask=m)
                return (ch + 1, cnt + pref[15])

            lax.while_loop(cond, wbody, (jnp.int32(1), cnt0))
        return carry
    lax.fori_loop(0, RPT, row_pass2, 0)

    @plsc.parallel_loop(0, NBCH, 1, unroll=4)
    def pair_chunk(c):
        p = c * 16 + lanes
        rloc = lax.shift_right_logical(p, 3)
        ig = row_base + rloc
        jj = cand_v[pl.ds(c * 16, 16)]
        valid = jj >= 0
        jc = jnp.maximum(jj, 0)
        t1 = [plsc.load_gather(tri_v[cc], [ig]) for cc in range(9)]
        t2 = [plsc.load_gather(tri_v[cc], [jc]) for cc in range(9)]
        ok = valid & _tri_tri_hit(t1, t2)
        plsc.store_scatter(oi_v, [p], ig, mask=ok)
        plsc.store_scatter(oj_v, [p], jj, mask=ok)

    pstart = row_base * K

    @pl.when(sid < NSUB - 1)
    def _():
        pltpu.sync_copy(oi_v.at[pl.ds(0, RPT * K)],
                        oi_hbm.at[cid, pl.ds(pstart, RPT * K)])
        pltpu.sync_copy(oj_v.at[pl.ds(0, RPT * K)],
                        oj_hbm.at[cid, pl.ds(pstart, RPT * K)])

    @pl.when(sid == NSUB - 1)
    def _():
        pltpu.sync_copy(oi_v.at[pl.ds(0, LASTR * K)],
                        oi_hbm.at[cid, pl.ds(pstart, LASTR * K)])
        pltpu.sync_copy(oj_v.at[pl.ds(0, LASTR * K)],
                        oj_hbm.at[cid, pl.ds(pstart, LASTR * K)])


@jax.jit
def _bvh_sc(tri_soa):
    mesh = plsc.VectorSubcoreMesh(core_axis_name="c", subcore_axis_name="s")
    fn = functools.partial(
        pl.kernel,
        out_type=(jax.ShapeDtypeStruct((B, F * K), jnp.int32),
                  jax.ShapeDtypeStruct((B, F * K), jnp.int32)),
        mesh=mesh,
        compiler_params=pltpu.CompilerParams(
            use_tc_tiling_on_sc=False, needs_layout_passes=False),
        scratch_types=(
            [pltpu.VMEM((FP,), jnp.float32) for _ in range(9)]
            + [pltpu.VMEM((FP,), jnp.float32) for _ in range(6)]
            + [pltpu.VMEM((NPAIR,), jnp.int32),
               pltpu.VMEM((NPAIR,), jnp.int32),
               pltpu.VMEM((NPAIR,), jnp.int32),
               pltpu.VMEM((320,), jnp.int32),
               pltpu.SemaphoreType.DMA]
        ),
    )(_bvh_body)
    return fn(tri_soa)


def kernel(triangles):
    tri_t = jnp.transpose(triangles, (0, 2, 3, 1)).reshape(B, 9, F)
    tri_p = jnp.pad(tri_t, ((0, 0), (0, 0), (0, FP - F)),
                    constant_values=PAD_VAL)
    oi, oj = _bvh_sc(tri_p.reshape(-1))
    return jnp.stack([oi, oj], axis=-1)
